# Initial kernel scaffold; baseline (speedup 1.0000x reference)
import jax
import jax.numpy as jnp
from jax.experimental import pallas as pl


def kernel(rows, cols, vals, weight):
    raise NotImplementedError("write your pallas kernel here")


# speedup vs baseline: 5.9219x; 5.9219x over previous
import dataclasses
import functools

import jax
import jax.numpy as jnp
from jax import lax
from jax.experimental import pallas as pl
from jax.experimental.pallas import tpu as pltpu
from jax.experimental.pallas import tpu_sc as plsc

N = 16384
D = 64
NC = 2
NS = 16
NW = NC * NS
E = 128
ROWS_PER_TILE = N // NS


def _sc_spmm(rows, cols, vals, weight):
    nnz = rows.shape[0]
    epw = pl.cdiv(nnz, NW * E) * E
    pad = epw * NW - nnz
    if pad:
        rows = jnp.concatenate([rows, jnp.zeros((pad,), rows.dtype)])
        cols = jnp.concatenate([cols, jnp.zeros((pad,), cols.dtype)])
        vals = jnp.concatenate([vals, jnp.zeros((pad,), vals.dtype)])

    mesh = plsc.VectorSubcoreMesh(core_axis_name="c", subcore_axis_name="s")
    cp = pltpu.CompilerParams()
    if "needs_layout_passes" in pltpu.CompilerParams.__dataclass_fields__:
        cp = dataclasses.replace(cp, needs_layout_passes=False)
    if "use_tc_tiling_on_sc" in pltpu.CompilerParams.__dataclass_fields__:
        cp = dataclasses.replace(cp, use_tc_tiling_on_sc=False)

    @functools.partial(
        pl.kernel,
        mesh=mesh,
        compiler_params=cp,
        out_type=jax.ShapeDtypeStruct((NC, N, D), jnp.float32),
        scratch_types=[
            pltpu.VMEM((E,), jnp.int32),
            pltpu.VMEM((E,), jnp.int32),
            pltpu.VMEM((E,), jnp.float32),
            pltpu.VMEM((E, D), jnp.float32),
            pltpu.VMEM_SHARED((N, D), jnp.float32),
            pltpu.SemaphoreType.DMA,
        ],
    )
    def spmm(rows_hbm, cols_hbm, vals_hbm, w_hbm, part_hbm,
             rows_v, cols_v, vals_v, g_v, acc, sem):
        cid = lax.axis_index("c")
        sid = lax.axis_index("s")

        @pl.loop(0, E)
        def _(r):
            for j in range(D // 16):
                g_v[r, pl.ds(j * 16, 16)] = jnp.zeros((16,), jnp.float32)

        for j in range(ROWS_PER_TILE // E):
            pltpu.sync_copy(g_v, acc.at[pl.ds(sid * ROWS_PER_TILE + j * E, E)])
        plsc.subcore_barrier()

        wid = sid * NC + cid
        base = wid * epw

        @pl.loop(0, epw, step=E)
        def _(c0):
            off = base + c0
            pltpu.sync_copy(rows_hbm.at[pl.ds(off, E)], rows_v)
            pltpu.sync_copy(cols_hbm.at[pl.ds(off, E)], cols_v)
            pltpu.sync_copy(vals_hbm.at[pl.ds(off, E)], vals_v)
            pltpu.async_copy(w_hbm.at[cols_v], g_v, sem).wait()

            @pl.loop(0, E)
            def _(e):
                vb = plsc.load_gather(vals_v, [jnp.full((16,), e, jnp.int32)])
                for j in range(D // 16):
                    g_v[e, pl.ds(j * 16, 16)] = g_v[e, pl.ds(j * 16, 16)] * vb

            pltpu.sync_copy(g_v, acc.at[rows_v], add=True)

        plsc.subcore_barrier()
        pltpu.sync_copy(
            acc.at[pl.ds(sid * ROWS_PER_TILE, ROWS_PER_TILE)],
            part_hbm.at[cid, pl.ds(sid * ROWS_PER_TILE, ROWS_PER_TILE)],
        )

    return spmm(rows, cols, vals, weight)


def _tc_combine(part):
    def body(p_ref, o_ref):
        o_ref[...] = p_ref[0] + p_ref[1]

    BR = 512
    return pl.pallas_call(
        body,
        out_shape=jax.ShapeDtypeStruct((N, D), jnp.float32),
        grid=(N // BR,),
        in_specs=[pl.BlockSpec((NC, BR, D), lambda i: (0, i, 0))],
        out_specs=pl.BlockSpec((BR, D), lambda i: (i, 0)),
    )(part)


def kernel(rows, cols, vals, weight):
    rows = rows.astype(jnp.int32)
    cols = cols.astype(jnp.int32)
    part = _sc_spmm(rows, cols, vals, weight)
    return _tc_combine(part)


# speedup vs baseline: 10.5288x; 1.7780x over previous
import dataclasses
import functools

import jax
import jax.numpy as jnp
from jax import lax
from jax.experimental import pallas as pl
from jax.experimental.pallas import tpu as pltpu
from jax.experimental.pallas import tpu_sc as plsc

N = 16384
D = 64
NC = 2
NS = 16
NW = NC * NS
Q = 128
NQ = 2
SB = Q * NQ
ROWS_PER_TILE = N // NS
NGB = 3
NIB = 4


def _sc_spmm(rows, cols, vals, weight):
    nnz = rows.shape[0]
    nsb = pl.cdiv(nnz, NW * SB)
    epw = nsb * SB
    pad = epw * NW - nnz
    if pad:
        rows = jnp.concatenate([rows, jnp.zeros((pad,), rows.dtype)])
        cols = jnp.concatenate([cols, jnp.zeros((pad,), cols.dtype)])
        vals = jnp.concatenate([vals, jnp.zeros((pad,), vals.dtype)])
    rows = rows.reshape(-1, Q)

    mesh = plsc.VectorSubcoreMesh(core_axis_name="c", subcore_axis_name="s")
    cp = pltpu.CompilerParams()
    if "needs_layout_passes" in pltpu.CompilerParams.__dataclass_fields__:
        cp = dataclasses.replace(cp, needs_layout_passes=False)
    if "use_tc_tiling_on_sc" in pltpu.CompilerParams.__dataclass_fields__:
        cp = dataclasses.replace(cp, use_tc_tiling_on_sc=False)

    scratch = (
        [pltpu.VMEM((NQ, Q), jnp.int32) for _ in range(NIB)]
        + [pltpu.VMEM((SB,), jnp.int32) for _ in range(NIB)]
        + [pltpu.VMEM((SB,), jnp.float32) for _ in range(NIB)]
        + [pltpu.VMEM((SB, D), jnp.float32) for _ in range(NGB)]
        + [pltpu.VMEM_SHARED((N, D), jnp.float32)]
        + [pltpu.SemaphoreType.DMA for _ in range(NIB)]
        + [pltpu.SemaphoreType.DMA for _ in range(NGB)]
        + [pltpu.SemaphoreType.DMA for _ in range(NGB)]
    )

    @functools.partial(
        pl.kernel,
        mesh=mesh,
        compiler_params=cp,
        out_type=jax.ShapeDtypeStruct((NC, N, D), jnp.float32),
        scratch_types=scratch,
    )
    def spmm(rows_hbm, cols_hbm, vals_hbm, w_hbm, part_hbm, *refs):
        rows_v = refs[0:NIB]
        cols_v = refs[NIB:2 * NIB]
        vals_v = refs[2 * NIB:3 * NIB]
        g_v = refs[3 * NIB:3 * NIB + NGB]
        acc = refs[3 * NIB + NGB]
        sem_i = refs[3 * NIB + NGB + 1:3 * NIB + NGB + 1 + NIB]
        sem_g = refs[3 * NIB + NGB + 1 + NIB:3 * NIB + NGB + 1 + NIB + NGB]
        sem_s = refs[3 * NIB + NGB + 1 + NIB + NGB:]

        cid = lax.axis_index("c")
        sid = lax.axis_index("s")

        @pl.loop(0, SB)
        def _(r):
            for j in range(D // 16):
                g_v[0][r, pl.ds(j * 16, 16)] = jnp.zeros((16,), jnp.float32)

        zbase = sid * ROWS_PER_TILE
        done = 0
        while done < ROWS_PER_TILE:
            step = min(SB, ROWS_PER_TILE - done)
            pltpu.sync_copy(g_v[0].at[pl.ds(0, step)],
                            acc.at[pl.ds(zbase + done, step)])
            done += step
        plsc.subcore_barrier()

        wid = sid * NC + cid
        base = wid * epw

        def issue_idx(s):
            b = s % NIB
            off = base + s * SB
            return [
                pltpu.async_copy(rows_hbm.at[pl.ds(off // Q, NQ)], rows_v[b],
                                 sem_i[b]),
                pltpu.async_copy(cols_hbm.at[pl.ds(off, SB)], cols_v[b],
                                 sem_i[b]),
                pltpu.async_copy(vals_hbm.at[pl.ds(off, SB)], vals_v[b],
                                 sem_i[b]),
            ]

        def issue_gather(s):
            b = s % NGB
            return pltpu.async_copy(w_hbm.at[cols_v[s % NIB]], g_v[b],
                                    sem_g[b])

        idx_c = {0: issue_idx(0)}
        if nsb > 1:
            idx_c[1] = issue_idx(1)
        for c in idx_c[0]:
            c.wait()
        g_c = {0: issue_gather(0)}
        s_c = {}

        for s in range(nsb):
            b = s % NGB
            if s - 2 >= 0:
                for c in s_c.pop(s - 2):
                    c.wait()
            if s + 1 < nsb:
                for c in idx_c.pop(s + 1):
                    c.wait()
                g_c[s + 1] = issue_gather(s + 1)
            g_c.pop(s).wait()

            vref = vals_v[s % NIB]
            gref = g_v[b]

            @pl.loop(0, SB, step=4)
            def _(e0):
                for u in range(4):
                    e = e0 + u
                    vb = plsc.load_gather(vref, [jnp.full((16,), e, jnp.int32)])
                    for j in range(D // 16):
                        gref[e, pl.ds(j * 16, 16)] = (
                            gref[e, pl.ds(j * 16, 16)] * vb)

            s_c[s] = [
                pltpu.async_copy(gref.at[pl.ds(q * Q, Q)],
                                 acc.at[rows_v[s % NIB].at[q]], sem_s[b],
                                 add=True)
                for q in range(NQ)
            ]
            if s + 2 < nsb:
                idx_c[s + 2] = issue_idx(s + 2)

        for cs in s_c.values():
            for c in cs:
                c.wait()

        plsc.subcore_barrier()
        pltpu.sync_copy(
            acc.at[pl.ds(sid * ROWS_PER_TILE, ROWS_PER_TILE)],
            part_hbm.at[cid, pl.ds(sid * ROWS_PER_TILE, ROWS_PER_TILE)],
        )

    return spmm(rows, cols, vals, weight)


def _tc_combine(part):
    def body(p_ref, o_ref):
        o_ref[...] = p_ref[0] + p_ref[1]

    BR = 512
    return pl.pallas_call(
        body,
        out_shape=jax.ShapeDtypeStruct((N, D), jnp.float32),
        grid=(N // BR,),
        in_specs=[pl.BlockSpec((NC, BR, D), lambda i: (0, i, 0))],
        out_specs=pl.BlockSpec((BR, D), lambda i: (i, 0)),
    )(part)


def kernel(rows, cols, vals, weight):
    rows = rows.astype(jnp.int32)
    cols = cols.astype(jnp.int32)
    part = _sc_spmm(rows, cols, vals, weight)
    return _tc_combine(part)
